# baseline (speedup 1.0000x reference)
import functools

import jax
import jax.numpy as jnp
from jax import lax
from jax.experimental import pallas as pl
from jax.experimental.pallas import tpu as pltpu, tpu_sc as plsc

_NC = 2
_NS = 16
_NW = _NC * _NS
_L = 16

_T = 16384
_D = 2048
_TPW = _T // _NW
_K = 16
_NCH = _TPW // _K


def _sc_body(embeds_hbm, mask_hbm, images_hbm, out_hbm,
             mask_v, mpos, upos, midx, uidx, imgidx,
             rowbuf0, rowbuf1, rowbuf2,
             sem_g0, sem_g1, sem_g2, sem_s0, sem_s1, sem_s2):
    sid = lax.axis_index("s")
    cid = lax.axis_index("c")
    wid = sid * _NC + cid
    tok0 = wid * _TPW

    pltpu.sync_copy(mask_hbm, mask_v)

    iota16 = lax.iota(jnp.int32, _L)

    def cbody(j, carry):
        cm, cu = carry
        mv = mask_v[pl.ds(tok0 + j * _L, _L)]
        tok = tok0 + j * _L + iota16
        inc = plsc.cumsum(mv)
        rank_m = cm + inc - mv
        rank_u = cu + iota16 + 1 - inc - (1 - mv)
        mb = mv > 0
        plsc.store_scatter(mpos, [rank_m], tok, mask=mb)
        plsc.store_scatter(upos, [rank_u], tok, mask=jnp.logical_not(mb))
        s = inc[_L - 1]
        return (cm + s, cu + (_L - s))

    m_w, u_w = lax.fori_loop(0, _TPW // _L, cbody,
                             (jnp.int32(0), jnp.int32(0)))

    u_last = jnp.maximum(u_w - 1, 0)

    def ubody(j, c):
        lu = jnp.minimum(j * _K + iota16, u_last)
        uidx[j, pl.ds(0, _L)] = plsc.load_gather(upos, [lu])
        return c

    lax.fori_loop(0, _NCH, ubody, jnp.int32(0))

    nu = (u_w + _K - 1) // _K
    nm = (m_w + _K - 1) // _K
    nm_full = m_w // _K
    nt = nu + nm

    bufs = (rowbuf0, rowbuf1, rowbuf2)
    gsems = (sem_g0, sem_g1, sem_g2)
    ssems = (sem_s0, sem_s1, sem_s2)

    def start_gather_u(j, b):
        @pl.when(j < nu)
        def _():
            pltpu.async_copy(embeds_hbm.at[uidx.at[j]], bufs[b], gsems[b])

    start_gather_u(0, 0)
    start_gather_u(1, 1)

    zero16 = jnp.zeros((_L,), jnp.int32)

    def pbody(i, accs):
        a0, a1, a2, a3 = accs
        base = i * (8 * _L)
        a0 = a0 + mask_v[pl.ds(base, _L)] + mask_v[pl.ds(base + 4 * _L, _L)]
        a1 = (a1 + mask_v[pl.ds(base + _L, _L)]
              + mask_v[pl.ds(base + 5 * _L, _L)])
        a2 = (a2 + mask_v[pl.ds(base + 2 * _L, _L)]
              + mask_v[pl.ds(base + 6 * _L, _L)])
        a3 = (a3 + mask_v[pl.ds(base + 3 * _L, _L)]
              + mask_v[pl.ds(base + 7 * _L, _L)])
        return (a0, a1, a2, a3)

    a0, a1, a2, a3 = lax.fori_loop(0, wid * (_TPW // (8 * _L)), pbody,
                                   (zero16, zero16, zero16, zero16))
    acc = (a0 + a1) + (a2 + a3)
    s_start = jnp.int32(0)
    for i in range(_L):
        s_start = s_start + acc[i]

    m_last = jnp.maximum(m_w - 1, 0)

    def mbody(j, c):
        lm = jnp.minimum(j * _K + iota16, m_last)
        midx[j, pl.ds(0, _L)] = plsc.load_gather(mpos, [lm])
        imgidx[j, pl.ds(0, _L)] = s_start + lm
        return c

    lax.fori_loop(0, _NCH, mbody, jnp.int32(0))

    def start_gather(j, b):
        start_gather_u(j, b)

        @pl.when(jnp.logical_and(j >= nu, j < nu + nm_full))
        def _():
            pltpu.async_copy(
                images_hbm.at[pl.ds(s_start + (j - nu) * _K, _K)],
                bufs[b], gsems[b])

        @pl.when(jnp.logical_and(j >= nu + nm_full, j < nt))
        def _():
            pltpu.async_copy(images_hbm.at[imgidx.at[j - nu]], bufs[b],
                             gsems[b])

    def start_scatter(j, b):
        @pl.when(j < nu)
        def _():
            pltpu.async_copy(bufs[b], out_hbm.at[uidx.at[j]], ssems[b])

        @pl.when(j >= nu)
        def _():
            pltpu.async_copy(bufs[b], out_hbm.at[midx.at[j - nu]], ssems[b])

    def wait_gather(b):
        pltpu.make_async_copy(images_hbm.at[imgidx.at[0]], bufs[b],
                              gsems[b]).wait()

    def wait_scatter(b):
        pltpu.make_async_copy(bufs[b], out_hbm.at[midx.at[0]],
                              ssems[b]).wait()

    for j, b in ((0, 0), (1, 1)):
        @pl.when(j >= nu)
        def _(j=j, b=b):
            start_gather(j, b)

    def ring(i, c):
        for b in (0, 1, 2):
            j = 3 * i + b

            @pl.when(j < nt)
            def _():
                wait_gather(b)
                start_scatter(j, b)

            @pl.when(jnp.logical_and(j >= 1, j <= nt))
            def _():
                wait_scatter((b + 2) % 3)

            @pl.when(j + 2 < nt)
            def _():
                start_gather(j + 2, (b + 2) % 3)
        return c

    lax.fori_loop(0, (nt + 2) // 3 + 1, ring, jnp.int32(0))


@jax.jit
def _scatter_block(embeds_flat, mask_i32, images):
    mesh = plsc.VectorSubcoreMesh(core_axis_name="c", subcore_axis_name="s")
    fn = functools.partial(
        pl.kernel,
        mesh=mesh,
        out_type=jax.ShapeDtypeStruct((_T, _D), jnp.float32),
        scratch_types=[
            pltpu.VMEM((_T,), jnp.int32),
            pltpu.VMEM((_TPW,), jnp.int32),
            pltpu.VMEM((_TPW,), jnp.int32),
            pltpu.VMEM((_NCH, _L), jnp.int32),
            pltpu.VMEM((_NCH, _L), jnp.int32),
            pltpu.VMEM((_NCH, _L), jnp.int32),
            pltpu.VMEM((_K, _D), jnp.float32),
            pltpu.VMEM((_K, _D), jnp.float32),
            pltpu.VMEM((_K, _D), jnp.float32),
            pltpu.SemaphoreType.DMA,
            pltpu.SemaphoreType.DMA,
            pltpu.SemaphoreType.DMA,
            pltpu.SemaphoreType.DMA,
            pltpu.SemaphoreType.DMA,
            pltpu.SemaphoreType.DMA,
        ],
        compiler_params=pltpu.CompilerParams(needs_layout_passes=False,
                                             use_tc_tiling_on_sc=False),
    )(_sc_body)
    return fn(embeds_flat, mask_i32, images)


def kernel(inputs_embeds, images_seq_mask, images_in_this_batch):
    shape = inputs_embeds.shape
    embeds_flat = inputs_embeds.reshape(_T, _D)
    mask_i32 = images_seq_mask.reshape(_T).astype(jnp.int32)
    out = _scatter_block(embeds_flat, mask_i32, images_in_this_batch)
    return out.reshape(shape)


# speedup vs baseline: 3.6564x; 3.6564x over previous
import functools

import jax
import jax.numpy as jnp
from jax import lax
from jax.experimental import pallas as pl
from jax.experimental.pallas import tpu as pltpu, tpu_sc as plsc

_NC = 2
_NS = 16
_NW = _NC * _NS
_L = 16

_T = 16384
_D = 2048
_TPW = _T // _NW
_K = 16
_NCH = _TPW // _K


def _sc_body(embeds_hbm, mask_hbm, images_hbm, out_hbm,
             mask_v, mpos, upos, midx, uidx, imgidx,
             rowbuf0, rowbuf1, rowbuf2,
             sem_g0, sem_g1, sem_g2, sem_s0, sem_s1, sem_s2):
    sid = lax.axis_index("s")
    cid = lax.axis_index("c")
    wid = sid * _NC + cid
    tok0 = wid * _TPW

    pltpu.sync_copy(mask_hbm, mask_v)

    iota16 = lax.iota(jnp.int32, _L)

    def cbody(j, carry):
        cm, cu = carry
        mv = mask_v[pl.ds(tok0 + j * _L, _L)]
        tok = tok0 + j * _L + iota16
        inc = plsc.cumsum(mv)
        rank_m = cm + inc - mv
        rank_u = cu + iota16 + 1 - inc - (1 - mv)
        mb = mv > 0
        plsc.store_scatter(mpos, [rank_m], tok, mask=mb)
        plsc.store_scatter(upos, [rank_u], tok, mask=jnp.logical_not(mb))
        s = inc[_L - 1]
        return (cm + s, cu + (_L - s))

    m_w, u_w = lax.fori_loop(0, _TPW // _L, cbody,
                             (jnp.int32(0), jnp.int32(0)))

    u_last = jnp.maximum(u_w - 1, 0)

    def ubody(j, c):
        lu = jnp.minimum(j * _K + iota16, u_last)
        uidx[j, pl.ds(0, _L)] = plsc.load_gather(upos, [lu])
        return c

    lax.fori_loop(0, _NCH, ubody, jnp.int32(0))

    nu = (u_w + _K - 1) // _K
    nm = (m_w + _K - 1) // _K
    nt = nu + nm

    bufs = (rowbuf0, rowbuf1, rowbuf2)
    gsems = (sem_g0, sem_g1, sem_g2)
    ssems = (sem_s0, sem_s1, sem_s2)

    def start_gather_u(j, b):
        @pl.when(j < nu)
        def _():
            pltpu.async_copy(embeds_hbm.at[uidx.at[j]], bufs[b], gsems[b])

    start_gather_u(0, 0)
    start_gather_u(1, 1)

    zero16 = jnp.zeros((_L,), jnp.int32)

    def pbody(i, accs):
        a0, a1, a2, a3 = accs
        base = i * (8 * _L)
        a0 = a0 + mask_v[pl.ds(base, _L)] + mask_v[pl.ds(base + 4 * _L, _L)]
        a1 = (a1 + mask_v[pl.ds(base + _L, _L)]
              + mask_v[pl.ds(base + 5 * _L, _L)])
        a2 = (a2 + mask_v[pl.ds(base + 2 * _L, _L)]
              + mask_v[pl.ds(base + 6 * _L, _L)])
        a3 = (a3 + mask_v[pl.ds(base + 3 * _L, _L)]
              + mask_v[pl.ds(base + 7 * _L, _L)])
        return (a0, a1, a2, a3)

    a0, a1, a2, a3 = lax.fori_loop(0, wid * (_TPW // (8 * _L)), pbody,
                                   (zero16, zero16, zero16, zero16))
    acc = (a0 + a1) + (a2 + a3)
    s_start = jnp.int32(0)
    for i in range(_L):
        s_start = s_start + acc[i]

    m_last = jnp.maximum(m_w - 1, 0)

    def mbody(j, c):
        lm = jnp.minimum(j * _K + iota16, m_last)
        midx[j, pl.ds(0, _L)] = plsc.load_gather(mpos, [lm])
        imgidx[j, pl.ds(0, _L)] = s_start + lm
        return c

    lax.fori_loop(0, _NCH, mbody, jnp.int32(0))

    def start_gather(j, b):
        start_gather_u(j, b)

        @pl.when(jnp.logical_and(j >= nu, j < nt))
        def _():
            pltpu.async_copy(images_hbm.at[imgidx.at[j - nu]], bufs[b],
                             gsems[b])

    def start_scatter(j, b):
        @pl.when(j < nu)
        def _():
            pltpu.async_copy(bufs[b], out_hbm.at[uidx.at[j]], ssems[b])

        @pl.when(j >= nu)
        def _():
            pltpu.async_copy(bufs[b], out_hbm.at[midx.at[j - nu]], ssems[b])

    def wait_gather(b):
        pltpu.make_async_copy(images_hbm.at[imgidx.at[0]], bufs[b],
                              gsems[b]).wait()

    def wait_scatter(b):
        pltpu.make_async_copy(bufs[b], out_hbm.at[midx.at[0]],
                              ssems[b]).wait()

    for j, b in ((0, 0), (1, 1)):
        @pl.when(j >= nu)
        def _(j=j, b=b):
            start_gather(j, b)

    def ring(i, c):
        for b in (0, 1, 2):
            j = 3 * i + b

            @pl.when(j < nt)
            def _():
                wait_gather(b)
                start_scatter(j, b)

            @pl.when(jnp.logical_and(j >= 1, j <= nt))
            def _():
                wait_scatter((b + 2) % 3)

            @pl.when(j + 2 < nt)
            def _():
                start_gather(j + 2, (b + 2) % 3)
        return c

    lax.fori_loop(0, (nt + 2) // 3 + 1, ring, jnp.int32(0))


@jax.jit
def _scatter_block(embeds_flat, mask_i32, images):
    mesh = plsc.VectorSubcoreMesh(core_axis_name="c", subcore_axis_name="s")
    fn = functools.partial(
        pl.kernel,
        mesh=mesh,
        out_type=jax.ShapeDtypeStruct((_T, _D), jnp.float32),
        scratch_types=[
            pltpu.VMEM((_T,), jnp.int32),
            pltpu.VMEM((_TPW,), jnp.int32),
            pltpu.VMEM((_TPW,), jnp.int32),
            pltpu.VMEM((_NCH, _L), jnp.int32),
            pltpu.VMEM((_NCH, _L), jnp.int32),
            pltpu.VMEM((_NCH, _L), jnp.int32),
            pltpu.VMEM((_K, _D), jnp.float32),
            pltpu.VMEM((_K, _D), jnp.float32),
            pltpu.VMEM((_K, _D), jnp.float32),
            pltpu.SemaphoreType.DMA,
            pltpu.SemaphoreType.DMA,
            pltpu.SemaphoreType.DMA,
            pltpu.SemaphoreType.DMA,
            pltpu.SemaphoreType.DMA,
            pltpu.SemaphoreType.DMA,
        ],
        compiler_params=pltpu.CompilerParams(needs_layout_passes=False),
    )(_sc_body)
    return fn(embeds_flat, mask_i32, images)


def kernel(inputs_embeds, images_seq_mask, images_in_this_batch):
    shape = inputs_embeds.shape
    embeds_flat = inputs_embeds.reshape(_T, _D)
    mask_i32 = images_seq_mask.reshape(_T).astype(jnp.int32)
    out = _scatter_block(embeds_flat, mask_i32, images_in_this_batch)
    return out.reshape(shape)
